# baseline (speedup 1.0000x reference)
import functools

import jax
import jax.numpy as jnp
from jax import lax
from jax.experimental import pallas as pl
from jax.experimental.pallas import tpu as pltpu
from jax.experimental.pallas import tpu_sc as plsc

_NC = 2
_NS = 16
_NW = _NC * _NS

_CHUNK = 100
_PADW = 16


def _sc_edge_pass(x_ext, d, src3, dst3, n_pad):
    n, de = x_ext.shape
    nch = src3.shape[1]
    slab = n_pad // _NS
    assert slab % 8 == 0

    mesh = plsc.VectorSubcoreMesh(core_axis_name="c", subcore_axis_name="s")

    @functools.partial(
        pl.kernel,
        out_type=jax.ShapeDtypeStruct((_NC, n_pad, de), jnp.float32),
        mesh=mesh,
        scratch_types=[
            pltpu.VMEM((nch, _CHUNK), jnp.int32),
            pltpu.VMEM((_CHUNK,), jnp.int32),
            pltpu.VMEM((_CHUNK,), jnp.int32),
            pltpu.VMEM((_CHUNK, de), jnp.float32),
            pltpu.VMEM((_CHUNK, de), jnp.float32),
            pltpu.VMEM_SHARED((n_pad, de), jnp.float32),
            pltpu.SemaphoreType.DMA,
            pltpu.SemaphoreType.DMA,
            pltpu.SemaphoreType.DMA,
            pltpu.SemaphoreType.DMA,
            pltpu.SemaphoreType.DMA,
            pltpu.SemaphoreType.DMA,
            pltpu.SemaphoreType.DMA,
        ],
        compiler_params=pltpu.CompilerParams(use_tc_tiling_on_sc=False),
    )
    def sc_kernel(x_hbm, src_hbm, dst_hbm, agg_hbm,
                  dst_all, srcv0, srcv1, rows0, rows1, agg_sh,
                  gsem0, gsem1, ssem0, ssem1, isem0, isem1, zsem):
        c = lax.axis_index("c")
        s = lax.axis_index("s")
        wid = c * _NS + s

        srcv = (srcv0, srcv1)
        rows = (rows0, rows1)
        gsem = (gsem0, gsem1)
        ssem = (ssem0, ssem1)
        isem = (isem0, isem1)

        row0 = pl.multiple_of(s * slab, 8)

        def zero_buf(i, carry):
            for j in range(de // 16):
                rows0[i, pl.ds(j * 16, 16)] = jnp.zeros((16,), jnp.float32)
            return carry

        lax.fori_loop(0, _CHUNK, zero_buf, 0)

        nfull, rem = divmod(slab, _CHUNK)
        zcopies = [(k * _CHUNK, _CHUNK) for k in range(nfull)]
        if rem:
            zcopies.append((nfull * _CHUNK, rem))
        for off, cnt in zcopies:
            pltpu.async_copy(rows0.at[pl.ds(0, cnt)],
                             agg_sh.at[pl.ds(row0 + off, cnt)], zsem)
        for off, cnt in zcopies:
            pltpu.make_async_copy(rows0.at[pl.ds(0, cnt)],
                                  agg_sh.at[pl.ds(row0 + off, cnt)],
                                  zsem).wait()

        pltpu.sync_copy(dst_hbm.at[wid], dst_all)
        pltpu.sync_copy(src_hbm.at[wid, 0], srcv0)
        plsc.subcore_barrier()

        def start_gather(p):
            pltpu.async_copy(x_hbm.at[srcv[p]], rows[p], gsem[p])

        def wait_gather(p):
            pltpu.make_async_copy(x_hbm.at[srcv[p]], rows[p], gsem[p]).wait()

        def start_scatter(ch, p):
            pltpu.async_copy(rows[p], agg_sh.at[dst_all.at[ch]], ssem[p],
                             add=True)

        def wait_scatter(p):
            pltpu.make_async_copy(
                rows[p], agg_sh.at[dst_all.at[0]], ssem[p]).wait()

        def start_idx(ch, p):
            pltpu.async_copy(src_hbm.at[wid, ch], srcv[p], isem[p])

        def wait_idx(p):
            pltpu.make_async_copy(
                src_hbm.at[wid, 0], srcv[p], isem[p]).wait()

        start_gather(0)
        start_idx(1, 1)
        wait_gather(0)
        wait_idx(1)
        start_gather(1)
        start_scatter(0, 0)
        start_idx(2, 0)

        def step(ch, p, q, gather_next, prefetch):
            wait_gather(p)
            wait_scatter(q)
            if gather_next is None:
                wait_idx(q)
                start_gather(q)
            elif gather_next is not False:
                @pl.when(gather_next)
                def _():
                    wait_idx(q)
                    start_gather(q)
            start_scatter(ch, p)
            if prefetch is None:
                start_idx(ch + 2, p)
            elif prefetch is not False:
                @pl.when(prefetch)
                def _():
                    start_idx(ch + 2, p)

        def pair(t, carry):
            c1 = 1 + 2 * t
            step(c1, 1, 0, None, c1 + 2 < nch)
            c2 = c1 + 1
            step(c2, 0, 1, c2 + 1 < nch, c2 + 2 < nch)
            return carry

        lax.fori_loop(0, (nch - 1) // 2, pair, 0)
        if (nch - 1) % 2 == 1:
            ct = nch - 1
            step(ct, ct % 2, 1 - ct % 2, False, False)
            wait_scatter(ct % 2)
        else:
            wait_scatter(0)
        plsc.subcore_barrier()

        pltpu.sync_copy(agg_sh.at[pl.ds(row0, slab)],
                        agg_hbm.at[c, pl.ds(row0, slab)])

    return sc_kernel(x_ext, src3, dst3)


def _tc_post(x, w_self, w_neigh, b, agg):
    n, d = x.shape
    de = agg.shape[2]
    blk = 2000

    def body(x_ref, ws_ref, wn_ref, b_ref, a0_ref, a1_ref, o_ref):
        a = a0_ref[0] + a1_ref[0]
        degs = jnp.maximum(a[:, d:d + 1], 1.0)
        h = a[:, :d] / degs
        o_ref[...] = (
            jnp.dot(x_ref[...], ws_ref[...], preferred_element_type=jnp.float32)
            + jnp.dot(h, wn_ref[...], preferred_element_type=jnp.float32)
            + b_ref[...]
        )

    return pl.pallas_call(
        body,
        grid=(n // blk,),
        in_specs=[
            pl.BlockSpec((blk, d), lambda i: (i, 0)),
            pl.BlockSpec((d, d), lambda i: (0, 0)),
            pl.BlockSpec((d, d), lambda i: (0, 0)),
            pl.BlockSpec((1, d), lambda i: (0, 0)),
            pl.BlockSpec((1, blk, de), lambda i: (0, i, 0)),
            pl.BlockSpec((1, blk, de), lambda i: (1, i, 0)),
        ],
        out_specs=pl.BlockSpec((blk, d), lambda i: (i, 0)),
        out_shape=jax.ShapeDtypeStruct((n, d), jnp.float32),
    )(x, w_self, w_neigh, b, agg, agg)


def kernel(x, edge_index, W_self, W_neigh, b):
    n, d = x.shape
    f = W_neigh.shape[1]

    e = edge_index.shape[1]
    epw = e // _NW
    nch = epw // _CHUNK
    src3 = edge_index[0].astype(jnp.int32).reshape(_NW, nch, _CHUNK)
    dst3 = edge_index[1].astype(jnp.int32).reshape(_NW, nch, _CHUNK)

    ext = jnp.zeros((n, _PADW), jnp.float32).at[:, 0].set(1.0)
    x_ext = jnp.concatenate([x, ext], axis=1)

    n_pad = ((n + _NS * 8 - 1) // (_NS * 8)) * (_NS * 8)
    agg = _sc_edge_pass(x_ext, d, src3, dst3, n_pad)
    return _tc_post(x, W_self, W_neigh, b.reshape(1, f), agg)


# speedup vs baseline: 1.2594x; 1.2594x over previous
import functools

import jax
import jax.numpy as jnp
from jax import lax
from jax.experimental import pallas as pl
from jax.experimental.pallas import tpu as pltpu
from jax.experimental.pallas import tpu_sc as plsc

_NC = 2
_NS = 16
_NW = _NC * _NS

_CHUNK = 100
_DEGW = 16


def _sc_edge_pass(x, src3, dst3, n_pad):
    n, d = x.shape
    nch = src3.shape[1]
    slab = n_pad // _NS
    assert slab % 8 == 0

    mesh = plsc.VectorSubcoreMesh(core_axis_name="c", subcore_axis_name="s")

    @functools.partial(
        pl.kernel,
        out_type=(jax.ShapeDtypeStruct((_NC, n_pad, d), jnp.float32),
                  jax.ShapeDtypeStruct((_NC, n_pad, _DEGW), jnp.float32)),
        mesh=mesh,
        scratch_types=[
            pltpu.VMEM((nch, _CHUNK), jnp.int32),
            pltpu.VMEM((_CHUNK,), jnp.int32),
            pltpu.VMEM((_CHUNK,), jnp.int32),
            pltpu.VMEM((_CHUNK, d), jnp.float32),
            pltpu.VMEM((_CHUNK, d), jnp.float32),
            pltpu.VMEM((_CHUNK, _DEGW), jnp.float32),
            pltpu.VMEM_SHARED((n_pad, d), jnp.float32),
            pltpu.VMEM_SHARED((n_pad, _DEGW), jnp.float32),
            pltpu.SemaphoreType.DMA,
            pltpu.SemaphoreType.DMA,
            pltpu.SemaphoreType.DMA,
            pltpu.SemaphoreType.DMA,
            pltpu.SemaphoreType.DMA,
            pltpu.SemaphoreType.DMA,
            pltpu.SemaphoreType.DMA,
            pltpu.SemaphoreType.DMA,
            pltpu.SemaphoreType.DMA,
        ],
        compiler_params=pltpu.CompilerParams(use_tc_tiling_on_sc=False),
    )
    def sc_kernel(x_hbm, src_hbm, dst_hbm, agg_hbm, deg_hbm,
                  dst_all, srcv0, srcv1, rows0, rows1, ones_v, agg_sh, deg_sh,
                  gsem0, gsem1, ssem0, ssem1, dsem0, dsem1, isem0, isem1,
                  zsem):
        c = lax.axis_index("c")
        s = lax.axis_index("s")
        wid = c * _NS + s

        srcv = (srcv0, srcv1)
        rows = (rows0, rows1)
        gsem = (gsem0, gsem1)
        ssem = (ssem0, ssem1)
        dsem = (dsem0, dsem1)
        isem = (isem0, isem1)

        row0 = pl.multiple_of(s * slab, 8)

        def zero_buf(i, carry):
            for j in range(d // 16):
                rows0[i, pl.ds(j * 16, 16)] = jnp.zeros((16,), jnp.float32)
            for j in range(_DEGW // 16):
                ones_v[i, pl.ds(j * 16, 16)] = jnp.zeros((16,), jnp.float32)
            return carry

        lax.fori_loop(0, _CHUNK, zero_buf, 0)

        nfull, rem = divmod(slab, _CHUNK)
        zcopies = []
        for k in range(nfull):
            zcopies.append((rows0, agg_sh, k * _CHUNK, _CHUNK))
            zcopies.append((ones_v, deg_sh, k * _CHUNK, _CHUNK))
        if rem:
            zcopies.append((rows0, agg_sh, nfull * _CHUNK, rem))
            zcopies.append((ones_v, deg_sh, nfull * _CHUNK, rem))
        for buf, sh, off, cnt in zcopies:
            pltpu.async_copy(buf.at[pl.ds(0, cnt)],
                             sh.at[pl.ds(row0 + off, cnt)], zsem)
        for buf, sh, off, cnt in zcopies:
            pltpu.make_async_copy(buf.at[pl.ds(0, cnt)],
                                  sh.at[pl.ds(row0 + off, cnt)], zsem).wait()

        def fill_ones(i, carry):
            for j in range(_DEGW // 16):
                ones_v[i, pl.ds(j * 16, 16)] = jnp.full((16,), 1.0,
                                                        jnp.float32)
            return carry

        lax.fori_loop(0, _CHUNK, fill_ones, 0)

        pltpu.sync_copy(dst_hbm.at[wid], dst_all)
        pltpu.sync_copy(src_hbm.at[wid, 0], srcv0)
        plsc.subcore_barrier()

        def start_gather(p):
            pltpu.async_copy(x_hbm.at[srcv[p]], rows[p], gsem[p])

        def wait_gather(p):
            pltpu.make_async_copy(x_hbm.at[srcv[p]], rows[p], gsem[p]).wait()

        def start_scatter(ch, p):
            pltpu.async_copy(rows[p], agg_sh.at[dst_all.at[ch]], ssem[p],
                             add=True)
            pltpu.async_copy(ones_v, deg_sh.at[dst_all.at[ch]], dsem[p],
                             add=True)

        def wait_scatter(p):
            pltpu.make_async_copy(
                rows[p], agg_sh.at[dst_all.at[0]], ssem[p]).wait()
            pltpu.make_async_copy(
                ones_v, deg_sh.at[dst_all.at[0]], dsem[p]).wait()

        def start_idx(ch, p):
            pltpu.async_copy(src_hbm.at[wid, ch], srcv[p], isem[p])

        def wait_idx(p):
            pltpu.make_async_copy(
                src_hbm.at[wid, 0], srcv[p], isem[p]).wait()

        start_gather(0)
        start_idx(1, 1)
        wait_gather(0)
        wait_idx(1)
        start_gather(1)
        start_scatter(0, 0)
        start_idx(2, 0)

        def step(ch, p, q, gather_next, prefetch):
            wait_gather(p)
            wait_scatter(q)
            if gather_next is None:
                wait_idx(q)
                start_gather(q)
            elif gather_next is not False:
                @pl.when(gather_next)
                def _():
                    wait_idx(q)
                    start_gather(q)
            start_scatter(ch, p)
            if prefetch is None:
                start_idx(ch + 2, p)
            elif prefetch is not False:
                @pl.when(prefetch)
                def _():
                    start_idx(ch + 2, p)

        def pair(t, carry):
            c1 = 1 + 2 * t
            step(c1, 1, 0, None, c1 + 2 < nch)
            c2 = c1 + 1
            step(c2, 0, 1, c2 + 1 < nch, c2 + 2 < nch)
            return carry

        lax.fori_loop(0, (nch - 1) // 2, pair, 0)
        if (nch - 1) % 2 == 1:
            ct = nch - 1
            step(ct, ct % 2, 1 - ct % 2, False, False)
            wait_scatter(ct % 2)
        else:
            wait_scatter(0)
        plsc.subcore_barrier()

        pltpu.sync_copy(agg_sh.at[pl.ds(row0, slab)],
                        agg_hbm.at[c, pl.ds(row0, slab)])
        pltpu.sync_copy(deg_sh.at[pl.ds(row0, slab)],
                        deg_hbm.at[c, pl.ds(row0, slab)])

    return sc_kernel(x, src3, dst3)


def _tc_post(x, w_self, w_neigh, b, agg, deg):
    n, d = x.shape
    blk = 2000

    def body(x_ref, ws_ref, wn_ref, b_ref, a0_ref, a1_ref, d0_ref, d1_ref,
             o_ref):
        degs = jnp.maximum((d0_ref[0] + d1_ref[0])[:, 0:1], 1.0)
        h = (a0_ref[0] + a1_ref[0]) / degs
        o_ref[...] = (
            jnp.dot(x_ref[...], ws_ref[...], preferred_element_type=jnp.float32)
            + jnp.dot(h, wn_ref[...], preferred_element_type=jnp.float32)
            + b_ref[...]
        )

    return pl.pallas_call(
        body,
        grid=(n // blk,),
        in_specs=[
            pl.BlockSpec((blk, d), lambda i: (i, 0)),
            pl.BlockSpec((d, d), lambda i: (0, 0)),
            pl.BlockSpec((d, d), lambda i: (0, 0)),
            pl.BlockSpec((1, d), lambda i: (0, 0)),
            pl.BlockSpec((1, blk, d), lambda i: (0, i, 0)),
            pl.BlockSpec((1, blk, d), lambda i: (1, i, 0)),
            pl.BlockSpec((1, blk, _DEGW), lambda i: (0, i, 0)),
            pl.BlockSpec((1, blk, _DEGW), lambda i: (1, i, 0)),
        ],
        out_specs=pl.BlockSpec((blk, d), lambda i: (i, 0)),
        out_shape=jax.ShapeDtypeStruct((n, d), jnp.float32),
    )(x, w_self, w_neigh, b, agg, agg, deg, deg)


def kernel(x, edge_index, W_self, W_neigh, b):
    n, d = x.shape
    f = W_neigh.shape[1]

    e = edge_index.shape[1]
    epw = e // _NW
    nch = epw // _CHUNK
    src3 = edge_index[0].astype(jnp.int32).reshape(_NW, nch, _CHUNK)
    dst3 = edge_index[1].astype(jnp.int32).reshape(_NW, nch, _CHUNK)

    n_pad = ((n + _NS * 8 - 1) // (_NS * 8)) * (_NS * 8)
    agg, deg = _sc_edge_pass(x, src3, dst3, n_pad)
    return _tc_post(x, W_self, W_neigh, b.reshape(1, f), agg, deg)
